# baseline (speedup 1.0000x reference)
import functools

import jax
import jax.numpy as jnp
import numpy as np
from jax.experimental import pallas as pl
from jax.experimental.pallas import tpu as pltpu

_N_ELEV = 64
_N_AZ = 360
_N_BINS = 50
_TGT_ELEV = 16
_EPS = 1e-08
_N_FREQS = _N_AZ // 2 + 1
_KPAD = 256
_BPAD = 128

_n = np.arange(_N_AZ)
_k = np.arange(_KPAD)
_ang = 2.0 * np.pi * ((_n[:, None] * _k[None, :]) % _N_AZ) / _N_AZ
_mask = (_k[None, :] < _N_FREQS).astype(np.float64)
_COS = np.asarray(np.cos(_ang) * _mask, dtype=np.float32)
_SIN = np.asarray(np.sin(_ang) * _mask, dtype=np.float32)


def _encoder_kernel(x_ref, c_ref, s_ref, a_ref, o_ref, *, rows):
    xb = x_ref[...]
    pooled = jnp.mean(xb, axis=1)

    hi = jax.lax.Precision.HIGHEST
    re = jax.lax.dot_general(
        pooled, c_ref[...], (((1,), (0,)), ((), ())),
        precision=hi, preferred_element_type=jnp.float32)
    im = jax.lax.dot_general(
        pooled, s_ref[...], (((1,), (0,)), ((), ())),
        precision=hi, preferred_element_type=jnp.float32)
    mag = jnp.sqrt(re * re + im * im)

    alpha = a_ref[0, 0]
    ji = jax.lax.broadcasted_iota(jnp.int32, (1, _KPAD), 1)
    j = ji.astype(jnp.float32)
    t = j * (1.0 / _N_BINS)
    denom = jnp.exp(alpha) - 1.0 + _EPS
    edges = (jnp.exp(alpha * t) - 1.0) / denom * _N_FREQS
    edge_valid = ji <= _N_BINS
    fii = jax.lax.broadcasted_iota(jnp.int32, (_KPAD, 1), 0)
    fi = fii.astype(jnp.float32)
    cnt = jnp.sum(
        jnp.where((edges <= fi) & edge_valid, 1.0, 0.0), axis=1,
        keepdims=True)
    assign = jnp.clip(cnt - 1.0, 0.0, _N_BINS - 1.0)
    bj = jax.lax.broadcasted_iota(jnp.int32, (1, _BPAD), 1).astype(
        jnp.float32)
    bmat = jnp.where((assign == bj) & (fii < _N_FREQS), 1.0, 0.0)

    hist = jax.lax.dot_general(
        mag, bmat, (((1,), (0,)), ((), ())),
        precision=hi, preferred_element_type=jnp.float32)

    rowsum = jnp.sum(hist, axis=1, keepdims=True)
    r0 = jax.lax.broadcasted_iota(jnp.int32, (rows, rows), 0) // _TGT_ELEV
    r1 = jax.lax.broadcasted_iota(jnp.int32, (rows, rows), 1) // _TGT_ELEV
    gmat = jnp.where(r0 == r1, 1.0, 0.0)
    stot = jax.lax.dot_general(
        gmat, rowsum, (((1,), (0,)), ((), ())),
        precision=hi, preferred_element_type=jnp.float32)

    out = jnp.where(stot > _EPS, hist / (stot + _EPS),
                    1.0 / (_TGT_ELEV * _N_BINS))
    o_ref[...] = out[:, :_N_BINS]


@jax.jit
def kernel(x, alpha):
    n = x.shape[0]
    xr = x.reshape(n * _TGT_ELEV, _N_ELEV // _TGT_ELEV, _N_AZ)
    samples_per_block = 32
    rows = samples_per_block * _TGT_ELEV
    grid = (n * _TGT_ELEV) // rows

    out = pl.pallas_call(
        functools.partial(_encoder_kernel, rows=rows),
        grid=(grid,),
        in_specs=[
            pl.BlockSpec((rows, _N_ELEV // _TGT_ELEV, _N_AZ),
                         lambda i: (i, 0, 0)),
            pl.BlockSpec((_N_AZ, _KPAD), lambda i: (0, 0)),
            pl.BlockSpec((_N_AZ, _KPAD), lambda i: (0, 0)),
            pl.BlockSpec((1, 1), lambda i: (0, 0)),
        ],
        out_specs=pl.BlockSpec((rows, _N_BINS), lambda i: (i, 0)),
        out_shape=jax.ShapeDtypeStruct((n * _TGT_ELEV, _N_BINS),
                                       jnp.float32),
        compiler_params=pltpu.CompilerParams(
            dimension_semantics=("arbitrary",)),
    )(xr, jnp.asarray(_COS), jnp.asarray(_SIN),
      jnp.asarray(alpha, jnp.float32).reshape(1, 1))

    return out.reshape(n, _TGT_ELEV * _N_BINS)


# speedup vs baseline: 2.6607x; 2.6607x over previous
import functools

import jax
import jax.numpy as jnp
import numpy as np
from jax.experimental import pallas as pl
from jax.experimental.pallas import tpu as pltpu

_N_ELEV = 64
_N_AZ = 360
_N_BINS = 50
_TGT_ELEV = 16
_EPS = 1e-08
_N_FREQS = _N_AZ // 2 + 1
_KPAD = 256
_BPAD = 128

_n = np.arange(_N_AZ)
_k = np.arange(_KPAD)
_ang = 2.0 * np.pi * ((_n[:, None] * _k[None, :]) % _N_AZ) / _N_AZ
_fmask = (_k[None, :] < _N_FREQS).astype(np.float64)
_COS = np.asarray(np.cos(_ang) * _fmask, dtype=jnp.bfloat16)
_SIN = np.asarray(np.sin(_ang) * _fmask, dtype=jnp.bfloat16)


def _pool_matrix(rows):
    r = np.arange(rows)
    c = np.arange(rows * 4)
    return np.asarray((c[None, :] // 4 == r[:, None]) * 0.25, jnp.bfloat16)


def _dot(a, b):
    return jax.lax.dot_general(a, b, (((1,), (0,)), ((), ())),
                               preferred_element_type=jnp.float32)


def _encoder_kernel(x_ref, p_ref, c_ref, s_ref, a_ref, o_ref, *, rows):
    xb16 = x_ref[...].astype(jnp.bfloat16)
    pooled = _dot(p_ref[...], xb16)
    p16 = pooled.astype(jnp.bfloat16)

    re = _dot(p16, c_ref[...])
    im = _dot(p16, s_ref[...])
    mag = jnp.sqrt(re * re + im * im)

    alpha = a_ref[0, 0]
    ji = jax.lax.broadcasted_iota(jnp.int32, (1, _KPAD), 1)
    j = ji.astype(jnp.float32)
    t = j * (1.0 / _N_BINS)
    denom = jnp.exp(alpha) - 1.0 + _EPS
    edges = (jnp.exp(alpha * t) - 1.0) / denom * _N_FREQS
    edge_valid = ji <= _N_BINS
    fii = jax.lax.broadcasted_iota(jnp.int32, (_KPAD, 1), 0)
    fi = fii.astype(jnp.float32)
    cnt = jnp.sum(
        jnp.where((edges <= fi) & edge_valid, 1.0, 0.0), axis=1,
        keepdims=True)
    assign = jnp.clip(cnt - 1.0, 0.0, _N_BINS - 1.0)
    bj = jax.lax.broadcasted_iota(jnp.int32, (1, _BPAD), 1).astype(
        jnp.float32)
    bmat = jnp.where((assign == bj) & (fii < _N_FREQS), 1.0,
                     0.0).astype(jnp.bfloat16)

    hist = _dot(mag.astype(jnp.bfloat16), bmat)

    rowsum = jnp.sum(hist, axis=1, keepdims=True)
    rs3 = rowsum.reshape(rows // _TGT_ELEV, _TGT_ELEV, 1)
    st = jnp.sum(rs3, axis=1, keepdims=True)
    stot = jnp.broadcast_to(st, rs3.shape).reshape(rows, 1)

    out = jnp.where(stot > _EPS, hist / (stot + _EPS),
                    1.0 / (_TGT_ELEV * _N_BINS))
    o_ref[...] = out[:, :_N_BINS]


@jax.jit
def kernel(x, alpha):
    n = x.shape[0]
    xr = x.reshape(n * _N_ELEV, _N_AZ)
    samples_per_block = 32
    rows = samples_per_block * _TGT_ELEV
    grid = (n * _TGT_ELEV) // rows

    out = pl.pallas_call(
        functools.partial(_encoder_kernel, rows=rows),
        grid=(grid,),
        in_specs=[
            pl.BlockSpec((rows * 4, _N_AZ), lambda i: (i, 0)),
            pl.BlockSpec((rows, rows * 4), lambda i: (0, 0)),
            pl.BlockSpec((_N_AZ, _KPAD), lambda i: (0, 0)),
            pl.BlockSpec((_N_AZ, _KPAD), lambda i: (0, 0)),
            pl.BlockSpec((1, 1), lambda i: (0, 0)),
        ],
        out_specs=pl.BlockSpec((rows, _N_BINS), lambda i: (i, 0)),
        out_shape=jax.ShapeDtypeStruct((n * _TGT_ELEV, _N_BINS),
                                       jnp.float32),
        compiler_params=pltpu.CompilerParams(
            dimension_semantics=("arbitrary",)),
    )(xr, _pool_matrix(rows), jnp.asarray(_COS), jnp.asarray(_SIN),
      jnp.asarray(alpha, jnp.float32).reshape(1, 1))

    return out.reshape(n, _TGT_ELEV * _N_BINS)


# speedup vs baseline: 10.1924x; 3.8307x over previous
import functools

import jax
import jax.numpy as jnp
import numpy as np
from jax.experimental import pallas as pl
from jax.experimental.pallas import tpu as pltpu

_N_ELEV = 64
_N_AZ = 360
_N_BINS = 50
_TGT_ELEV = 16
_EPS = 1e-08
_N_FREQS = _N_AZ // 2 + 1
_KPAD = 256
_BPAD = 128

_n = np.arange(_N_AZ)
_k = np.arange(_KPAD)
_ang = 2.0 * np.pi * ((_k[:, None] * _n[None, :]) % _N_AZ) / _N_AZ
_fmask = (_k[:, None] < _N_FREQS).astype(np.float64)
_COS_T = np.asarray(np.cos(_ang) * _fmask, dtype=jnp.bfloat16)
_SIN_T = np.asarray(np.sin(_ang) * _fmask, dtype=jnp.bfloat16)


def _dot(a, b):
    return jax.lax.dot_general(a, b, (((1,), (0,)), ((), ())),
                               preferred_element_type=jnp.float32)


def _encoder_kernel(x_ref, c_ref, s_ref, a_ref, o_ref, *, bc):
    alpha = a_ref[0, 0]
    ji = jax.lax.broadcasted_iota(jnp.int32, (64, 1), 0)
    j = ji.astype(jnp.float32)
    t = j * (1.0 / _N_BINS)
    denom = jnp.exp(alpha) - 1.0 + _EPS
    edges = (jnp.exp(alpha * t) - 1.0) / denom * _N_FREQS
    edge_valid = ji <= _N_BINS
    fii = jax.lax.broadcasted_iota(jnp.int32, (1, _KPAD), 1)
    fi = fii.astype(jnp.float32)
    cnt = jnp.sum(
        jnp.where((edges <= fi) & edge_valid, 1.0, 0.0), axis=0,
        keepdims=True)
    assign = jnp.clip(cnt - 1.0, 0.0, _N_BINS - 1.0)
    bj = jax.lax.broadcasted_iota(jnp.int32, (_BPAD, 1), 0).astype(
        jnp.float32)
    bmat_t = jnp.where((assign == bj) & (fii < _N_FREQS), 1.0,
                       0.0).astype(jnp.bfloat16)

    ct = c_ref[...]
    st = s_ref[...]
    tot = jnp.zeros((1, bc), jnp.float32)
    for e in range(_TGT_ELEV):
        xe = x_ref[4 * e] + x_ref[4 * e + 1] + x_ref[4 * e + 2] \
            + x_ref[4 * e + 3]
        p16 = (xe * 0.25).astype(jnp.bfloat16)
        re = _dot(ct, p16)
        im = _dot(st, p16)
        mag = jnp.sqrt(re * re + im * im)
        hist = _dot(bmat_t, mag.astype(jnp.bfloat16))
        tot = tot + jnp.sum(hist, axis=0, keepdims=True)
        o_ref[e * _N_BINS:(e + 1) * _N_BINS, :] = hist[:_N_BINS, :]

    inv = 1.0 / (tot + _EPS)
    o_ref[...] = jnp.where(tot > _EPS, o_ref[...] * inv,
                           1.0 / (_TGT_ELEV * _N_BINS))


@jax.jit
def kernel(x, alpha):
    n = x.shape[0]
    xt = jnp.transpose(x, (1, 2, 0))
    bc = 128
    grid = n // bc

    out_t = pl.pallas_call(
        functools.partial(_encoder_kernel, bc=bc),
        grid=(grid,),
        in_specs=[
            pl.BlockSpec((_N_ELEV, _N_AZ, bc), lambda i: (0, 0, i)),
            pl.BlockSpec((_KPAD, _N_AZ), lambda i: (0, 0)),
            pl.BlockSpec((_KPAD, _N_AZ), lambda i: (0, 0)),
            pl.BlockSpec((1, 1), lambda i: (0, 0)),
        ],
        out_specs=pl.BlockSpec((_TGT_ELEV * _N_BINS, bc), lambda i: (0, i)),
        out_shape=jax.ShapeDtypeStruct((_TGT_ELEV * _N_BINS, n),
                                       jnp.float32),
        compiler_params=pltpu.CompilerParams(
            dimension_semantics=("arbitrary",)),
    )(xt, jnp.asarray(_COS_T), jnp.asarray(_SIN_T),
      jnp.asarray(alpha, jnp.float32).reshape(1, 1))

    return jnp.transpose(out_t, (1, 0))


# speedup vs baseline: 11.4570x; 1.1241x over previous
import functools

import jax
import jax.numpy as jnp
import numpy as np
from jax.experimental import pallas as pl
from jax.experimental.pallas import tpu as pltpu

_N_ELEV = 64
_N_AZ = 360
_N_BINS = 50
_TGT_ELEV = 16
_EPS = 1e-08
_N_FREQS = _N_AZ // 2 + 1
_KPAD = 256
_BPAD = 128

_n = np.arange(_N_AZ)
_k = np.arange(_KPAD)
_ang = 2.0 * np.pi * ((_k[:, None] * _n[None, :]) % _N_AZ) / _N_AZ
_fmask = (_k[:, None] < _N_FREQS).astype(np.float64)
_COS_T = np.asarray(np.cos(_ang) * _fmask, dtype=jnp.bfloat16)
_SIN_T = np.asarray(np.sin(_ang) * _fmask, dtype=jnp.bfloat16)


def _dot(a, b):
    return jax.lax.dot_general(a, b, (((1,), (0,)), ((), ())),
                               preferred_element_type=jnp.float32)


def _bin_matrix_t(alpha):
    ji = jax.lax.broadcasted_iota(jnp.int32, (64, 1), 0)
    t = ji.astype(jnp.float32) * (1.0 / _N_BINS)
    denom = jnp.exp(alpha) - 1.0 + _EPS
    edges = (jnp.exp(alpha * t) - 1.0) / denom * _N_FREQS
    edge_valid = ji <= _N_BINS
    fii = jax.lax.broadcasted_iota(jnp.int32, (1, _KPAD), 1)
    fi = fii.astype(jnp.float32)
    cnt = jnp.sum(
        jnp.where((edges <= fi) & edge_valid, 1.0, 0.0), axis=0,
        keepdims=True)
    assign = jnp.clip(cnt - 1.0, 0.0, _N_BINS - 1.0)
    bj = jax.lax.broadcasted_iota(jnp.int32, (_BPAD, 1), 0).astype(
        jnp.float32)
    return jnp.where((assign == bj) & (fii < _N_FREQS), 1.0,
                     0.0).astype(jnp.bfloat16)


def _hist_kernel(x_ref, c_ref, s_ref, a_ref, o_ref, t_ref, *, bc, ep):
    bmat_t = _bin_matrix_t(a_ref[0, 0])
    ct = c_ref[...]
    st = s_ref[...]
    partial = jnp.zeros((1, bc), jnp.float32)
    for k in range(ep):
        xe = x_ref[4 * k] + x_ref[4 * k + 1] + x_ref[4 * k + 2] \
            + x_ref[4 * k + 3]
        p16 = (xe * 0.25).astype(jnp.bfloat16)
        re = _dot(ct, p16)
        im = _dot(st, p16)
        mag = jnp.sqrt(re * re + im * im)
        hist = _dot(bmat_t, mag.astype(jnp.bfloat16))
        partial = partial + jnp.sum(hist, axis=0, keepdims=True)
        o_ref[k * _N_BINS:(k + 1) * _N_BINS, :] = hist[:_N_BINS, :]
    t_ref[0] = partial


def _norm_kernel(h_ref, t_ref, o_ref):
    tot = jnp.sum(t_ref[...], axis=0)
    inv = 1.0 / (tot + _EPS)
    o_ref[...] = jnp.where(tot > _EPS, h_ref[...] * inv,
                           1.0 / (_TGT_ELEV * _N_BINS))


@jax.jit
def kernel(x, alpha):
    n = x.shape[0]
    xt = jnp.transpose(x, (1, 2, 0))
    bc = 512
    nj = 4
    er = _N_ELEV // nj
    ep = er // 4

    hist_t, tots = pl.pallas_call(
        functools.partial(_hist_kernel, bc=bc, ep=ep),
        grid=(n // bc, nj),
        in_specs=[
            pl.BlockSpec((er, _N_AZ, bc), lambda i, j: (j, 0, i)),
            pl.BlockSpec((_KPAD, _N_AZ), lambda i, j: (0, 0)),
            pl.BlockSpec((_KPAD, _N_AZ), lambda i, j: (0, 0)),
            pl.BlockSpec((1, 1), lambda i, j: (0, 0)),
        ],
        out_specs=[
            pl.BlockSpec((ep * _N_BINS, bc), lambda i, j: (j, i)),
            pl.BlockSpec((1, 1, bc), lambda i, j: (j, 0, i)),
        ],
        out_shape=[
            jax.ShapeDtypeStruct((_TGT_ELEV * _N_BINS, n), jnp.float32),
            jax.ShapeDtypeStruct((nj, 1, n), jnp.float32),
        ],
        compiler_params=pltpu.CompilerParams(
            dimension_semantics=("arbitrary", "arbitrary")),
    )(xt, jnp.asarray(_COS_T), jnp.asarray(_SIN_T),
      jnp.asarray(alpha, jnp.float32).reshape(1, 1))

    bc2 = 512
    out_t = pl.pallas_call(
        _norm_kernel,
        grid=(n // bc2,),
        in_specs=[
            pl.BlockSpec((_TGT_ELEV * _N_BINS, bc2), lambda i: (0, i)),
            pl.BlockSpec((nj, 1, bc2), lambda i: (0, 0, i)),
        ],
        out_specs=pl.BlockSpec((_TGT_ELEV * _N_BINS, bc2),
                               lambda i: (0, i)),
        out_shape=jax.ShapeDtypeStruct((_TGT_ELEV * _N_BINS, n),
                                       jnp.float32),
        compiler_params=pltpu.CompilerParams(
            dimension_semantics=("arbitrary",)),
    )(hist_t, tots)

    return jnp.transpose(out_t, (1, 0))
